# Initial kernel scaffold; baseline (speedup 1.0000x reference)
import jax
import jax.numpy as jnp
from jax.experimental import pallas as pl


def kernel(token_embeddings, uzman_embeddings, W1, b1, W2, b2, onbellek_durumu):
    raise NotImplementedError("write your pallas kernel here")


# speedup vs baseline: 2.0713x; 2.0713x over previous
import functools

import jax
import jax.numpy as jnp
from jax.experimental import pallas as pl
from jax.experimental.pallas import tpu as pltpu

E = 8
TOP_K = 2
LOKAL = 7
D = 768
H = 1024
N = 2048
H_CHUNK = 256
NEG = -3.0e38


def _routing(x, ue, onb):
    sims = jax.lax.dot_general(x, ue, (((1,), (1,)), ((), ())))
    xn = jnp.sqrt(jnp.sum(x * x, axis=1, keepdims=True))
    en = jnp.sqrt(jnp.sum(ue * ue, axis=1, keepdims=True))
    tok_n = x / (xn + 1e-8)
    exp_n = ue / (en + 1e-8)
    cos = jax.lax.dot_general(tok_n, exp_n, (((1,), (1,)), ((), ())))
    t = cos + 0.1 * onb
    eidx = jax.lax.broadcasted_iota(jnp.int32, (N, E), 1)
    m = jnp.min(sims, axis=1, keepdims=True)
    excl = jnp.max(jnp.where(sims == m, eidx, -1), axis=1, keepdims=True)
    t = jnp.where(eidx == excl, NEG, t)
    t0 = jnp.max(t, axis=1, keepdims=True)
    i0 = jnp.min(jnp.where(t == t0, eidx, E), axis=1, keepdims=True)
    t1m = jnp.where(eidx == i0, NEG, t)
    t1 = jnp.max(t1m, axis=1, keepdims=True)
    i1 = jnp.min(jnp.where(t1m == t1, eidx, E), axis=1, keepdims=True)
    ed = jnp.exp(t1 - t0)
    s = 1.0 + ed
    w0 = 1.0 / s
    w1 = ed / s
    combine = jnp.where(eidx == i0, w0, 0.0) + jnp.where(eidx == i1, w1, 0.0)
    return combine, i0, i1, w0, w1


def _moe_kernel(x_ref, ue_ref, w1_ref, b1_ref, w2_ref, b2_ref, onb_ref,
                out_ref, idx_ref, w_ref, comb_ref):
    e = pl.program_id(0)

    @pl.when(e == 0)
    def _():
        combine, i0, i1, w0, w1 = _routing(x_ref[...], ue_ref[...],
                                           onb_ref[...])
        comb_ref[...] = combine.T
        idx_ref[...] = jnp.concatenate([i0, i1], axis=1)
        w_ref[...] = jnp.concatenate([w0, w1], axis=1)

    x = x_ref[...]
    b2 = b2_ref[0]

    def body(c, y):
        h = jax.lax.dot_general(
            x, w1_ref[0, :, pl.ds(c * H_CHUNK, H_CHUNK)],
            (((1,), (0,)), ((), ())), preferred_element_type=jnp.float32)
        h = h + b1_ref[0, :, pl.ds(c * H_CHUNK, H_CHUNK)]
        a = 0.5 * h * (1.0 + jax.lax.erf(h * 0.7071067811865476))
        y = y + jax.lax.dot_general(
            a, w2_ref[0, pl.ds(c * H_CHUNK, H_CHUNK), :],
            (((1,), (0,)), ((), ())), preferred_element_type=jnp.float32)
        return y

    y = jax.lax.fori_loop(0, H // H_CHUNK, body,
                          jnp.zeros((N, D), jnp.float32))
    y = y + b2
    c_row = comb_ref[pl.ds(e, 1), :]
    contrib = c_row.T * y

    @pl.when(e == 0)
    def _():
        out_ref[...] = contrib

    @pl.when(e != 0)
    def _():
        out_ref[...] = out_ref[...] + contrib


@jax.jit
def kernel(token_embeddings, uzman_embeddings, W1, b1, W2, b2,
           onbellek_durumu):
    b, s, d = token_embeddings.shape
    x = token_embeddings.reshape(-1, d)
    onb = onbellek_durumu.reshape(1, E)
    b1r = b1.reshape(E, 1, H)
    b2r = b2.reshape(E, 1, D)
    out, idx, w = pl.pallas_call(
        _moe_kernel,
        grid=(E,),
        in_specs=[
            pl.BlockSpec((N, D), lambda e: (0, 0)),
            pl.BlockSpec((E, D), lambda e: (0, 0)),
            pl.BlockSpec((1, D, H), lambda e: (e, 0, 0)),
            pl.BlockSpec((1, 1, H), lambda e: (e, 0, 0)),
            pl.BlockSpec((1, H, D), lambda e: (e, 0, 0)),
            pl.BlockSpec((1, 1, D), lambda e: (e, 0, 0)),
            pl.BlockSpec((1, E), lambda e: (0, 0)),
        ],
        out_specs=[
            pl.BlockSpec((N, D), lambda e: (0, 0)),
            pl.BlockSpec((N, TOP_K), lambda e: (0, 0)),
            pl.BlockSpec((N, TOP_K), lambda e: (0, 0)),
        ],
        out_shape=[
            jax.ShapeDtypeStruct((N, D), jnp.float32),
            jax.ShapeDtypeStruct((N, TOP_K), jnp.int32),
            jax.ShapeDtypeStruct((N, TOP_K), jnp.float32),
        ],
        scratch_shapes=[pltpu.VMEM((E, N), jnp.float32)],
    )(x, uzman_embeddings, W1, b1r, W2, b2r, onb)
    return (out.reshape(b, s, d), idx.reshape(b, s, TOP_K),
            w.reshape(b, s, TOP_K))
